# baseline (speedup 1.0000x reference)
import functools

import jax
import jax.numpy as jnp
from jax import lax
from jax.experimental import pallas as pl
from jax.experimental.pallas import tpu as pltpu
from jax.experimental.pallas import tpu_sc as plsc

N = 10000
E = 320000
D = 128
C = 10
P = 4096

NPAD = 10240
NTILES = 32
EP = E // NTILES
CH = 128
NCHT = 80
EPP = NCHT * CH
IG = 4
GN = NCHT // IG
ROWS_PER_TILE = NPAD // 16
BM = 1024

_f32 = jnp.float32


def _k1_body(h_ref, ws_ref, wn_ref, b_ref, hs_ref, hn_ref):
    hb = h_ref[...]
    hs_ref[...] = jnp.dot(hb, ws_ref[...], preferred_element_type=_f32) + b_ref[...]
    hn_ref[...] = jnp.dot(hb, wn_ref[...], preferred_element_type=_f32)


def _k1(hpad, ws, wn, b):
    grid = (NPAD // BM,)
    return pl.pallas_call(
        _k1_body,
        grid=grid,
        in_specs=[
            pl.BlockSpec((BM, D), lambda i: (i, 0)),
            pl.BlockSpec((D, D), lambda i: (0, 0)),
            pl.BlockSpec((D, D), lambda i: (0, 0)),
            pl.BlockSpec((1, D), lambda i: (0, 0)),
        ],
        out_specs=[
            pl.BlockSpec((BM, D), lambda i: (i, 0)),
            pl.BlockSpec((BM, D), lambda i: (i, 0)),
        ],
        out_shape=[
            jax.ShapeDtypeStruct((NPAD, D), _f32),
            jax.ShapeDtypeStruct((NPAD, D), _f32),
        ],
    )(hpad, ws, wn, b)


def _dinv_col(degp_blk):
    deg8 = degp_blk[0] + degp_blk[1]
    dinv8 = 1.0 / jnp.maximum(deg8, 1.0)
    lane_of_row = lax.broadcasted_iota(jnp.int32, (BM, D), 0) % D
    lane_ids = lax.broadcasted_iota(jnp.int32, (BM, D), 1)
    onehot = (lane_of_row == lane_ids).astype(_f32)
    r = lax.dot_general(onehot, dinv8, (((1,), (1,)), ((), ())),
                        preferred_element_type=_f32)
    grp = lax.broadcasted_iota(jnp.int32, (BM, 8), 0) // D
    sel = (grp == lax.broadcasted_iota(jnp.int32, (BM, 8), 1)).astype(_f32)
    return jnp.sum(r * sel, axis=1, keepdims=True)


def _k2_body(hs0_ref, a0_ref, a1_ref, degp_ref, ws_ref, wn_ref, b_ref,
             hs1_ref, hn1_ref, dinv_ref):
    dinv = _dinv_col(degp_ref[...])
    h1 = jnp.maximum(hs0_ref[...] + (a0_ref[...] + a1_ref[...]) * dinv, 0.0)
    hs1_ref[...] = jnp.dot(h1, ws_ref[...], preferred_element_type=_f32) + b_ref[...]
    hn1_ref[...] = jnp.dot(h1, wn_ref[...], preferred_element_type=_f32)
    dinv_ref[...] = jnp.broadcast_to(dinv, (BM, D))


def _k2(hs0, agg_a, agg_b, degp, ws, wn, b):
    grid = (NPAD // BM,)
    return pl.pallas_call(
        _k2_body,
        grid=grid,
        in_specs=[
            pl.BlockSpec((BM, D), lambda i: (i, 0)),
            pl.BlockSpec((BM, D), lambda i: (i, 0)),
            pl.BlockSpec((BM, D), lambda i: (i, 0)),
            pl.BlockSpec((2, 8, D), lambda i: (0, i, 0)),
            pl.BlockSpec((D, D), lambda i: (0, 0)),
            pl.BlockSpec((D, D), lambda i: (0, 0)),
            pl.BlockSpec((1, D), lambda i: (0, 0)),
        ],
        out_specs=[
            pl.BlockSpec((BM, D), lambda i: (i, 0)),
            pl.BlockSpec((BM, D), lambda i: (i, 0)),
            pl.BlockSpec((BM, D), lambda i: (i, 0)),
        ],
        out_shape=[
            jax.ShapeDtypeStruct((NPAD, D), _f32),
            jax.ShapeDtypeStruct((NPAD, D), _f32),
            jax.ShapeDtypeStruct((NPAD, D), _f32),
        ],
    )(hs0, agg_a, agg_b, degp, ws, wn, b)


def _k3_body(hs1_ref, a0_ref, a1_ref, dinv_ref, w2_ref, b2_ref, z_ref):
    agg = a0_ref[...] + a1_ref[...]
    h2 = jnp.maximum(hs1_ref[...] + agg * dinv_ref[...], 0.0)
    z_ref[...] = jnp.dot(h2, w2_ref[...], preferred_element_type=_f32) + b2_ref[...]


def _k3(hs1, agg_a, agg_b, dinvb, w2p, b2h):
    grid = (NPAD // BM,)
    return pl.pallas_call(
        _k3_body,
        grid=grid,
        in_specs=[
            pl.BlockSpec((BM, D), lambda i: (i, 0)),
            pl.BlockSpec((BM, D), lambda i: (i, 0)),
            pl.BlockSpec((BM, D), lambda i: (i, 0)),
            pl.BlockSpec((BM, D), lambda i: (i, 0)),
            pl.BlockSpec((D, D), lambda i: (0, 0)),
            pl.BlockSpec((1, D), lambda i: (0, 0)),
        ],
        out_specs=pl.BlockSpec((BM, D), lambda i: (i, 0)),
        out_shape=jax.ShapeDtypeStruct((NPAD, D), _f32),
    )(hs1, agg_a, agg_b, dinvb, w2p, b2h)


def _make_sagg(compute_deg):
    mesh = plsc.VectorSubcoreMesh(core_axis_name="c", subcore_axis_name="s",
                                  num_cores=2, num_subcores=16)
    out_type = [
        jax.ShapeDtypeStruct((NPAD, D), _f32),
        jax.ShapeDtypeStruct((NPAD, D), _f32),
    ]
    scratch = [
        pltpu.VMEM((2, GN, CH), jnp.int32),
        pltpu.VMEM((2, GN, CH), jnp.int32),
        pltpu.VMEM((CH, D), _f32),
        pltpu.VMEM((CH, D), _f32),
        pltpu.VMEM_SHARED((NPAD + 8, D), _f32),
        pltpu.SemaphoreType.DMA,
        pltpu.SemaphoreType.DMA,
        pltpu.SemaphoreType.DMA,
        pltpu.SemaphoreType.DMA,
    ]
    if compute_deg:
        out_type = out_type + [jax.ShapeDtypeStruct((2 * NPAD,), _f32)]
        scratch = scratch + [
            pltpu.VMEM((CH,), _f32),
            pltpu.VMEM_SHARED((NPAD + 8,), _f32),
        ]

    @functools.partial(pl.kernel, out_type=tuple(out_type), mesh=mesh,
                       scratch_types=scratch)
    def sagg(hn_hbm, src_hbm, dst_hbm, zz_hbm, *rest):
        if compute_deg:
            (zzdeg_hbm, out_a, out_b, out_deg,
             srcv, dstv, rows0, rows1, acc, gsem0, gsem1, isem0, isem1,
             onesv, accdeg) = rest
        else:
            (out_a, out_b,
             srcv, dstv, rows0, rows1, acc, gsem0, gsem1, isem0, isem1) = rest
        cid = lax.axis_index("c")
        sid = lax.axis_index("s")
        wid = cid * 16 + sid
        row0 = sid * ROWS_PER_TILE
        isems = (isem0, isem1)

        def idx_load(g):
            b = g % 2
            pltpu.async_copy(src_hbm.at[wid, g], srcv.at[b], isems[b])
            pltpu.async_copy(dst_hbm.at[wid, g], dstv.at[b], isems[b])

        def idx_wait(g):
            b = g % 2
            pltpu.make_async_copy(src_hbm.at[wid, g], srcv.at[b], isems[b]).wait()
            pltpu.make_async_copy(dst_hbm.at[wid, g], dstv.at[b], isems[b]).wait()

        idx_load(0)
        pltpu.sync_copy(zz_hbm, acc.at[pl.ds(row0, ROWS_PER_TILE)])
        if compute_deg:
            for j in range(CH // 16):
                onesv[pl.ds(j * 16, 16)] = jnp.ones((16,), _f32)

            @pl.when(sid == 0)
            def _():
                pltpu.sync_copy(zzdeg_hbm, accdeg.at[pl.ds(0, NPAD)])
        plsc.subcore_barrier()

        for g in range(IG):
            b = g % 2
            idx_wait(g)
            if g + 1 < IG:
                idx_load(g + 1)
            pltpu.async_copy(hn_hbm.at[srcv.at[b, 0]], rows0, gsem0)

            def body(i, carry, b=b):
                c = 2 * i
                pltpu.async_copy(hn_hbm.at[srcv.at[b, c + 1]], rows1, gsem1)
                if compute_deg:
                    pltpu.sync_copy(onesv, accdeg.at[dstv.at[b, c]], add=True)
                pltpu.make_async_copy(hn_hbm.at[srcv.at[b, c]], rows0, gsem0).wait()
                pltpu.sync_copy(rows0, acc.at[dstv.at[b, c]], add=True)
                pltpu.async_copy(hn_hbm.at[srcv.at[b, c + 2]], rows0, gsem0)
                if compute_deg:
                    pltpu.sync_copy(onesv, accdeg.at[dstv.at[b, c + 1]], add=True)
                pltpu.make_async_copy(hn_hbm.at[srcv.at[b, c + 1]], rows1, gsem1).wait()
                pltpu.sync_copy(rows1, acc.at[dstv.at[b, c + 1]], add=True)
                return carry

            lax.fori_loop(0, (GN - 2) // 2, body, 0)
            pltpu.async_copy(hn_hbm.at[srcv.at[b, GN - 1]], rows1, gsem1)
            if compute_deg:
                pltpu.sync_copy(onesv, accdeg.at[dstv.at[b, GN - 2]], add=True)
            pltpu.make_async_copy(hn_hbm.at[srcv.at[b, GN - 2]], rows0, gsem0).wait()
            pltpu.sync_copy(rows0, acc.at[dstv.at[b, GN - 2]], add=True)
            if compute_deg:
                pltpu.sync_copy(onesv, accdeg.at[dstv.at[b, GN - 1]], add=True)
            pltpu.make_async_copy(hn_hbm.at[srcv.at[b, GN - 1]], rows1, gsem1).wait()
            pltpu.sync_copy(rows1, acc.at[dstv.at[b, GN - 1]], add=True)

        plsc.subcore_barrier()

        @pl.when(cid == 0)
        def _():
            pltpu.sync_copy(acc.at[pl.ds(row0, ROWS_PER_TILE)],
                            out_a.at[pl.ds(row0, ROWS_PER_TILE)])

        @pl.when(cid == 1)
        def _():
            pltpu.sync_copy(acc.at[pl.ds(row0, ROWS_PER_TILE)],
                            out_b.at[pl.ds(row0, ROWS_PER_TILE)])

        if compute_deg:
            @pl.when(sid == 1)
            def _():
                pltpu.sync_copy(accdeg.at[pl.ds(0, NPAD)],
                                out_deg.at[pl.ds(cid * NPAD, NPAD)])

    return sagg


_sagg_deg = _make_sagg(True)
_sagg = _make_sagg(False)


def _make_readout():
    mesh = plsc.VectorSubcoreMesh(core_axis_name="c", subcore_axis_name="s",
                                  num_cores=2, num_subcores=16)
    rows = P // NTILES

    @functools.partial(
        pl.kernel,
        out_type=jax.ShapeDtypeStruct((P, D), _f32),
        mesh=mesh,
        scratch_types=[
            pltpu.VMEM((rows,), jnp.int32),
            pltpu.VMEM((rows,), jnp.int32),
            pltpu.VMEM((rows, D), _f32),
            pltpu.VMEM((rows, D), _f32),
            pltpu.SemaphoreType.DMA,
            pltpu.SemaphoreType.DMA,
        ],
    )
    def readout(z_hbm, x1_hbm, x2_hbm, out_hbm, i1, i2, r1, r2, s1, s2):
        cid = lax.axis_index("c")
        sid = lax.axis_index("s")
        base = (cid * 16 + sid) * rows
        pltpu.sync_copy(x1_hbm.at[pl.ds(base, rows)], i1)
        pltpu.sync_copy(x2_hbm.at[pl.ds(base, rows)], i2)
        d1 = pltpu.async_copy(z_hbm.at[i1], r1, s1)
        d2 = pltpu.async_copy(z_hbm.at[i2], r2, s2)
        d1.wait()
        d2.wait()

        def add_row(i, carry):
            for j in range(D // 16):
                sl = pl.ds(j * 16, 16)
                r1[i, sl] = r1[i, sl] + r2[i, sl]
            return carry

        lax.fori_loop(0, rows, add_row, 0)
        pltpu.sync_copy(r1, out_hbm.at[pl.ds(base, rows)])

    return readout


_readout = _make_readout()


def kernel(h, edge_index, x1, x2, W_self0, W_neigh0, b0,
           W_self1, W_neigh1, b1, W2, b2):
    src2 = edge_index[0].reshape(NTILES, EP)
    dst2 = edge_index[1].reshape(NTILES, EP)
    src3d = jnp.pad(src2, ((0, 0), (0, EPP - EP))).reshape(NTILES, IG, GN, CH)
    dst3d = jnp.pad(dst2, ((0, 0), (0, EPP - EP)),
                    constant_values=NPAD).reshape(NTILES, IG, GN, CH)
    hpad = jnp.pad(h, ((0, NPAD - N), (0, 0)))

    b0r = b0.reshape(1, D)
    b1r = b1.reshape(1, D)
    w2p = jnp.pad(W2, ((0, 0), (0, D - C)))
    b2h = jnp.pad(0.5 * b2, (0, D - C)).reshape(1, D)
    zz = jnp.zeros((ROWS_PER_TILE, D), _f32)
    zzdeg = jnp.zeros((NPAD,), _f32)

    hs0, hn0 = _k1(hpad, W_self0, W_neigh0, b0r)
    agg_a0, agg_b0, deg = _sagg_deg(hn0, src3d, dst3d, zz, zzdeg)
    degp = deg.reshape(2, NPAD // D, D)
    hs1, hn1, dinvb = _k2(hs0, agg_a0, agg_b0, degp, W_self1, W_neigh1, b1r)
    agg_a1, agg_b1 = _sagg(hn1, src3d, dst3d, zz)
    z = _k3(hs1, agg_a1, agg_b1, dinvb, w2p, b2h)
    outp = _readout(z, x1, x2)
    return outp[:, :C]


# speedup vs baseline: 2.6275x; 2.6275x over previous
import functools

import jax
import jax.numpy as jnp
from jax import lax
from jax.experimental import pallas as pl
from jax.experimental.pallas import tpu as pltpu
from jax.experimental.pallas import tpu_sc as plsc

N = 10000
E = 320000
D = 128
C = 10
P = 4096

NPAD = 10240
NTILES = 32
EP = E // NTILES
CH = 80
NCHT = EP // CH
IG = 5
GN = NCHT // IG
ROWS_PER_TILE = NPAD // 16
BM = 1024

_f32 = jnp.float32


def _k1_body(h_ref, ws_ref, wn_ref, b_ref, hs_ref, hn_ref):
    hb = h_ref[...]
    hs_ref[...] = jnp.dot(hb, ws_ref[...], preferred_element_type=_f32) + b_ref[...]
    hn_ref[...] = jnp.dot(hb, wn_ref[...], preferred_element_type=_f32)


def _k1(hpad, ws, wn, b):
    grid = (NPAD // BM,)
    return pl.pallas_call(
        _k1_body,
        grid=grid,
        in_specs=[
            pl.BlockSpec((BM, D), lambda i: (i, 0)),
            pl.BlockSpec((D, D), lambda i: (0, 0)),
            pl.BlockSpec((D, D), lambda i: (0, 0)),
            pl.BlockSpec((1, D), lambda i: (0, 0)),
        ],
        out_specs=[
            pl.BlockSpec((BM, D), lambda i: (i, 0)),
            pl.BlockSpec((BM, D), lambda i: (i, 0)),
        ],
        out_shape=[
            jax.ShapeDtypeStruct((NPAD, D), _f32),
            jax.ShapeDtypeStruct((NPAD, D), _f32),
        ],
    )(hpad, ws, wn, b)


def _dinv_col(degp_blk):
    deg8 = degp_blk[0] + degp_blk[1]
    dinv8 = 1.0 / jnp.maximum(deg8, 1.0)
    lane_of_row = lax.broadcasted_iota(jnp.int32, (BM, D), 0) % D
    lane_ids = lax.broadcasted_iota(jnp.int32, (BM, D), 1)
    onehot = (lane_of_row == lane_ids).astype(_f32)
    r = lax.dot_general(onehot, dinv8, (((1,), (1,)), ((), ())),
                        preferred_element_type=_f32)
    grp = lax.broadcasted_iota(jnp.int32, (BM, 8), 0) // D
    sel = (grp == lax.broadcasted_iota(jnp.int32, (BM, 8), 1)).astype(_f32)
    return jnp.sum(r * sel, axis=1, keepdims=True)


def _k2_body(hs0_ref, a0_ref, a1_ref, degp_ref, ws_ref, wn_ref, b_ref,
             hs1_ref, hn1_ref, dinv_ref):
    dinv = _dinv_col(degp_ref[...])
    h1 = jnp.maximum(hs0_ref[...] + (a0_ref[...] + a1_ref[...]) * dinv, 0.0)
    hs1_ref[...] = jnp.dot(h1, ws_ref[...], preferred_element_type=_f32) + b_ref[...]
    hn1_ref[...] = jnp.dot(h1, wn_ref[...], preferred_element_type=_f32)
    dinv_ref[...] = jnp.broadcast_to(dinv, (BM, D))


def _k2(hs0, agg_a, agg_b, degp, ws, wn, b):
    grid = (NPAD // BM,)
    return pl.pallas_call(
        _k2_body,
        grid=grid,
        in_specs=[
            pl.BlockSpec((BM, D), lambda i: (i, 0)),
            pl.BlockSpec((BM, D), lambda i: (i, 0)),
            pl.BlockSpec((BM, D), lambda i: (i, 0)),
            pl.BlockSpec((2, 8, D), lambda i: (0, i, 0)),
            pl.BlockSpec((D, D), lambda i: (0, 0)),
            pl.BlockSpec((D, D), lambda i: (0, 0)),
            pl.BlockSpec((1, D), lambda i: (0, 0)),
        ],
        out_specs=[
            pl.BlockSpec((BM, D), lambda i: (i, 0)),
            pl.BlockSpec((BM, D), lambda i: (i, 0)),
            pl.BlockSpec((BM, D), lambda i: (i, 0)),
        ],
        out_shape=[
            jax.ShapeDtypeStruct((NPAD, D), _f32),
            jax.ShapeDtypeStruct((NPAD, D), _f32),
            jax.ShapeDtypeStruct((NPAD, D), _f32),
        ],
    )(hs0, agg_a, agg_b, degp, ws, wn, b)


def _k3_body(hs1_ref, a0_ref, a1_ref, dinv_ref, w2_ref, b2_ref, z_ref):
    agg = a0_ref[...] + a1_ref[...]
    h2 = jnp.maximum(hs1_ref[...] + agg * dinv_ref[...], 0.0)
    z_ref[...] = jnp.dot(h2, w2_ref[...], preferred_element_type=_f32) + b2_ref[...]


def _k3(hs1, agg_a, agg_b, dinvb, w2p, b2h):
    grid = (NPAD // BM,)
    return pl.pallas_call(
        _k3_body,
        grid=grid,
        in_specs=[
            pl.BlockSpec((BM, D), lambda i: (i, 0)),
            pl.BlockSpec((BM, D), lambda i: (i, 0)),
            pl.BlockSpec((BM, D), lambda i: (i, 0)),
            pl.BlockSpec((BM, D), lambda i: (i, 0)),
            pl.BlockSpec((D, D), lambda i: (0, 0)),
            pl.BlockSpec((1, D), lambda i: (0, 0)),
        ],
        out_specs=pl.BlockSpec((BM, D), lambda i: (i, 0)),
        out_shape=jax.ShapeDtypeStruct((NPAD, D), _f32),
    )(hs1, agg_a, agg_b, dinvb, w2p, b2h)


def _make_sagg(compute_deg):
    mesh = plsc.VectorSubcoreMesh(core_axis_name="c", subcore_axis_name="s",
                                  num_cores=2, num_subcores=16)
    out_type = [
        jax.ShapeDtypeStruct((NPAD, D), _f32),
        jax.ShapeDtypeStruct((NPAD, D), _f32),
    ]
    scratch = [
        pltpu.VMEM((2, GN, CH), jnp.int32),
        pltpu.VMEM((2, GN, CH), jnp.int32),
        pltpu.VMEM((CH, D), _f32),
        pltpu.VMEM((CH, D), _f32),
        pltpu.VMEM_SHARED((NPAD, D), _f32),
        pltpu.SemaphoreType.DMA,
        pltpu.SemaphoreType.DMA,
        pltpu.SemaphoreType.DMA,
        pltpu.SemaphoreType.DMA,
    ]
    if compute_deg:
        out_type = out_type + [jax.ShapeDtypeStruct((2 * NPAD,), _f32)]
        scratch = scratch + [
            pltpu.VMEM((CH,), _f32),
            pltpu.VMEM_SHARED((NPAD,), _f32),
        ]

    @functools.partial(pl.kernel, out_type=tuple(out_type), mesh=mesh,
                       scratch_types=scratch)
    def sagg(hn_hbm, src_hbm, dst_hbm, zz_hbm, *rest):
        if compute_deg:
            (zzdeg_hbm, out_a, out_b, out_deg,
             srcv, dstv, rows0, rows1, acc, gsem0, gsem1, isem0, isem1,
             onesv, accdeg) = rest
        else:
            (out_a, out_b,
             srcv, dstv, rows0, rows1, acc, gsem0, gsem1, isem0, isem1) = rest
        cid = lax.axis_index("c")
        sid = lax.axis_index("s")
        wid = cid * 16 + sid
        row0 = sid * ROWS_PER_TILE
        isems = (isem0, isem1)

        def idx_load(g):
            b = g % 2
            pltpu.async_copy(src_hbm.at[wid, g], srcv.at[b], isems[b])
            pltpu.async_copy(dst_hbm.at[wid, g], dstv.at[b], isems[b])

        def idx_wait(g):
            b = g % 2
            pltpu.make_async_copy(src_hbm.at[wid, g], srcv.at[b], isems[b]).wait()
            pltpu.make_async_copy(dst_hbm.at[wid, g], dstv.at[b], isems[b]).wait()

        idx_load(0)
        pltpu.sync_copy(zz_hbm, acc.at[pl.ds(row0, ROWS_PER_TILE)])
        if compute_deg:
            for j in range(CH // 16):
                onesv[pl.ds(j * 16, 16)] = jnp.ones((16,), _f32)

            @pl.when(sid == 0)
            def _():
                pltpu.sync_copy(zzdeg_hbm, accdeg)
        plsc.subcore_barrier()

        for g in range(IG):
            b = g % 2
            idx_wait(g)
            if g + 1 < IG:
                idx_load(g + 1)
            pltpu.async_copy(hn_hbm.at[srcv.at[b, 0]], rows0, gsem0)

            def body(i, carry, b=b):
                c = 2 * i
                pltpu.async_copy(hn_hbm.at[srcv.at[b, c + 1]], rows1, gsem1)
                if compute_deg:
                    pltpu.sync_copy(onesv, accdeg.at[dstv.at[b, c]], add=True)
                pltpu.make_async_copy(hn_hbm.at[srcv.at[b, c]], rows0, gsem0).wait()
                pltpu.sync_copy(rows0, acc.at[dstv.at[b, c]], add=True)
                pltpu.async_copy(hn_hbm.at[srcv.at[b, c + 2]], rows0, gsem0)
                if compute_deg:
                    pltpu.sync_copy(onesv, accdeg.at[dstv.at[b, c + 1]], add=True)
                pltpu.make_async_copy(hn_hbm.at[srcv.at[b, c + 1]], rows1, gsem1).wait()
                pltpu.sync_copy(rows1, acc.at[dstv.at[b, c + 1]], add=True)
                return carry

            lax.fori_loop(0, (GN - 1) // 2, body, 0)
            pltpu.make_async_copy(hn_hbm.at[srcv.at[b, GN - 1]], rows0, gsem0).wait()
            pltpu.sync_copy(rows0, acc.at[dstv.at[b, GN - 1]], add=True)
            if compute_deg:
                pltpu.sync_copy(onesv, accdeg.at[dstv.at[b, GN - 1]], add=True)

        plsc.subcore_barrier()

        @pl.when(cid == 0)
        def _():
            pltpu.sync_copy(acc.at[pl.ds(row0, ROWS_PER_TILE)],
                            out_a.at[pl.ds(row0, ROWS_PER_TILE)])

        @pl.when(cid == 1)
        def _():
            pltpu.sync_copy(acc.at[pl.ds(row0, ROWS_PER_TILE)],
                            out_b.at[pl.ds(row0, ROWS_PER_TILE)])

        if compute_deg:
            @pl.when(sid == 1)
            def _():
                pltpu.sync_copy(accdeg, out_deg.at[pl.ds(cid * NPAD, NPAD)])

    return sagg


_sagg_deg = _make_sagg(True)
_sagg = _make_sagg(False)


def _make_readout():
    mesh = plsc.VectorSubcoreMesh(core_axis_name="c", subcore_axis_name="s",
                                  num_cores=2, num_subcores=16)
    rows = P // NTILES

    @functools.partial(
        pl.kernel,
        out_type=jax.ShapeDtypeStruct((P, D), _f32),
        mesh=mesh,
        scratch_types=[
            pltpu.VMEM((rows,), jnp.int32),
            pltpu.VMEM((rows,), jnp.int32),
            pltpu.VMEM((rows, D), _f32),
            pltpu.VMEM((rows, D), _f32),
            pltpu.SemaphoreType.DMA,
            pltpu.SemaphoreType.DMA,
        ],
    )
    def readout(z_hbm, x1_hbm, x2_hbm, out_hbm, i1, i2, r1, r2, s1, s2):
        cid = lax.axis_index("c")
        sid = lax.axis_index("s")
        base = (cid * 16 + sid) * rows
        pltpu.sync_copy(x1_hbm.at[pl.ds(base, rows)], i1)
        pltpu.sync_copy(x2_hbm.at[pl.ds(base, rows)], i2)
        d1 = pltpu.async_copy(z_hbm.at[i1], r1, s1)
        d2 = pltpu.async_copy(z_hbm.at[i2], r2, s2)
        d1.wait()
        d2.wait()

        def add_row(i, carry):
            for j in range(D // 16):
                sl = pl.ds(j * 16, 16)
                r1[i, sl] = r1[i, sl] + r2[i, sl]
            return carry

        lax.fori_loop(0, rows, add_row, 0)
        pltpu.sync_copy(r1, out_hbm.at[pl.ds(base, rows)])

    return readout


_readout = _make_readout()


def kernel(h, edge_index, x1, x2, W_self0, W_neigh0, b0,
           W_self1, W_neigh1, b1, W2, b2):
    src3d = edge_index[0].reshape(NTILES, IG, GN, CH)
    dst3d = edge_index[1].reshape(NTILES, IG, GN, CH)
    hpad = jnp.pad(h, ((0, NPAD - N), (0, 0)))

    b0r = b0.reshape(1, D)
    b1r = b1.reshape(1, D)
    w2p = jnp.pad(W2, ((0, 0), (0, D - C)))
    b2h = jnp.pad(0.5 * b2, (0, D - C)).reshape(1, D)
    zz = jnp.zeros((ROWS_PER_TILE, D), _f32)
    zzdeg = jnp.zeros((NPAD,), _f32)

    hs0, hn0 = _k1(hpad, W_self0, W_neigh0, b0r)
    agg_a0, agg_b0, deg = _sagg_deg(hn0, src3d, dst3d, zz, zzdeg)
    degp = deg.reshape(2, NPAD // D, D)
    hs1, hn1, dinvb = _k2(hs0, agg_a0, agg_b0, degp, W_self1, W_neigh1, b1r)
    agg_a1, agg_b1 = _sagg(hn1, src3d, dst3d, zz)
    z = _k3(hs1, agg_a1, agg_b1, dinvb, w2p, b2h)
    outp = _readout(z, x1, x2)
    return outp[:, :C]


# speedup vs baseline: 2.6654x; 1.0144x over previous
import functools

import jax
import jax.numpy as jnp
from jax import lax
from jax.experimental import pallas as pl
from jax.experimental.pallas import tpu as pltpu
from jax.experimental.pallas import tpu_sc as plsc

N = 10000
E = 320000
D = 128
C = 10
P = 4096

NPAD = 10240
NTILES = 32
EP = E // NTILES
CH = 80
NCHT = EP // CH
IG = 5
GN = NCHT // IG
ROWS_PER_TILE = NPAD // 16
BM = 1024

_f32 = jnp.float32


def _k1_body(h_ref, wn_ref, hn_ref):
    hn_ref[...] = jnp.dot(h_ref[...], wn_ref[...], preferred_element_type=_f32)


def _k1(h, wn):
    grid = (NPAD // BM,)
    return pl.pallas_call(
        _k1_body,
        grid=grid,
        in_specs=[
            pl.BlockSpec((BM, D), lambda i: (i, 0)),
            pl.BlockSpec((D, D), lambda i: (0, 0)),
        ],
        out_specs=pl.BlockSpec((BM, D), lambda i: (i, 0)),
        out_shape=jax.ShapeDtypeStruct((NPAD, D), _f32),
    )(h, wn)


def _dinv_col(degp_blk):
    deg8 = degp_blk[0] + degp_blk[1]
    dinv8 = 1.0 / jnp.maximum(deg8, 1.0)
    lane_of_row = lax.broadcasted_iota(jnp.int32, (BM, D), 0) % D
    lane_ids = lax.broadcasted_iota(jnp.int32, (BM, D), 1)
    onehot = (lane_of_row == lane_ids).astype(_f32)
    r = lax.dot_general(onehot, dinv8, (((1,), (1,)), ((), ())),
                        preferred_element_type=_f32)
    grp = lax.broadcasted_iota(jnp.int32, (BM, 8), 0) // D
    sel = (grp == lax.broadcasted_iota(jnp.int32, (BM, 8), 1)).astype(_f32)
    return jnp.sum(r * sel, axis=1, keepdims=True)


def _k2_body(h_ref, a0_ref, a1_ref, degp_ref, ws_ref, b_ref, wn_ref,
             h1_ref, hn1_ref):
    dinv = _dinv_col(degp_ref[...])
    hs0 = jnp.dot(h_ref[...], ws_ref[...], preferred_element_type=_f32) + b_ref[...]
    h1 = jnp.maximum(hs0 + (a0_ref[...] + a1_ref[...]) * dinv, 0.0)
    h1_ref[...] = h1
    hn1_ref[...] = jnp.dot(h1, wn_ref[...], preferred_element_type=_f32)


def _k2(h, agg_a, agg_b, degp, ws, b, wn):
    grid = (NPAD // BM,)
    return pl.pallas_call(
        _k2_body,
        grid=grid,
        in_specs=[
            pl.BlockSpec((BM, D), lambda i: (i, 0)),
            pl.BlockSpec((BM, D), lambda i: (i, 0)),
            pl.BlockSpec((BM, D), lambda i: (i, 0)),
            pl.BlockSpec((2, 8, D), lambda i: (0, i, 0)),
            pl.BlockSpec((D, D), lambda i: (0, 0)),
            pl.BlockSpec((1, D), lambda i: (0, 0)),
            pl.BlockSpec((D, D), lambda i: (0, 0)),
        ],
        out_specs=[
            pl.BlockSpec((BM, D), lambda i: (i, 0)),
            pl.BlockSpec((BM, D), lambda i: (i, 0)),
        ],
        out_shape=[
            jax.ShapeDtypeStruct((NPAD, D), _f32),
            jax.ShapeDtypeStruct((NPAD, D), _f32),
        ],
    )(h, agg_a, agg_b, degp, ws, b, wn)


def _k3_body(h1_ref, a0_ref, a1_ref, degp_ref, ws_ref, b_ref, w2_ref, b2_ref,
             z_ref):
    dinv = _dinv_col(degp_ref[...])
    hs1 = jnp.dot(h1_ref[...], ws_ref[...], preferred_element_type=_f32) + b_ref[...]
    h2 = jnp.maximum(hs1 + (a0_ref[...] + a1_ref[...]) * dinv, 0.0)
    z_ref[...] = jnp.dot(h2, w2_ref[...], preferred_element_type=_f32) + b2_ref[...]


def _k3(h1, agg_a, agg_b, degp, ws, b, w2p, b2h):
    grid = (NPAD // BM,)
    return pl.pallas_call(
        _k3_body,
        grid=grid,
        in_specs=[
            pl.BlockSpec((BM, D), lambda i: (i, 0)),
            pl.BlockSpec((BM, D), lambda i: (i, 0)),
            pl.BlockSpec((BM, D), lambda i: (i, 0)),
            pl.BlockSpec((2, 8, D), lambda i: (0, i, 0)),
            pl.BlockSpec((D, D), lambda i: (0, 0)),
            pl.BlockSpec((1, D), lambda i: (0, 0)),
            pl.BlockSpec((D, D), lambda i: (0, 0)),
            pl.BlockSpec((1, D), lambda i: (0, 0)),
        ],
        out_specs=pl.BlockSpec((BM, D), lambda i: (i, 0)),
        out_shape=jax.ShapeDtypeStruct((NPAD, D), _f32),
    )(h1, agg_a, agg_b, degp, ws, b, w2p, b2h)


def _make_sagg(compute_deg):
    mesh = plsc.VectorSubcoreMesh(core_axis_name="c", subcore_axis_name="s",
                                  num_cores=2, num_subcores=16)
    out_type = [
        jax.ShapeDtypeStruct((NPAD, D), _f32),
        jax.ShapeDtypeStruct((NPAD, D), _f32),
    ]
    scratch = [
        pltpu.VMEM((2, GN, CH), jnp.int32),
        pltpu.VMEM((2, GN, CH), jnp.int32),
        pltpu.VMEM((CH, D), _f32),
        pltpu.VMEM((CH, D), _f32),
        pltpu.VMEM_SHARED((NPAD, D), _f32),
        pltpu.SemaphoreType.DMA,
        pltpu.SemaphoreType.DMA,
        pltpu.SemaphoreType.DMA,
        pltpu.SemaphoreType.DMA,
    ]
    if compute_deg:
        out_type = out_type + [jax.ShapeDtypeStruct((2 * NPAD,), _f32)]
        scratch = scratch + [
            pltpu.VMEM((CH,), _f32),
            pltpu.VMEM_SHARED((NPAD,), _f32),
        ]

    @functools.partial(pl.kernel, out_type=tuple(out_type), mesh=mesh,
                       scratch_types=scratch)
    def sagg(hn_hbm, src_hbm, dst_hbm, zz_hbm, *rest):
        if compute_deg:
            (zzdeg_hbm, out_a, out_b, out_deg,
             srcv, dstv, rows0, rows1, acc, gsem0, gsem1, isem0, isem1,
             onesv, accdeg) = rest
        else:
            (out_a, out_b,
             srcv, dstv, rows0, rows1, acc, gsem0, gsem1, isem0, isem1) = rest
        cid = lax.axis_index("c")
        sid = lax.axis_index("s")
        wid = cid * 16 + sid
        row0 = sid * ROWS_PER_TILE
        isems = (isem0, isem1)

        def idx_load(g):
            b = g % 2
            pltpu.async_copy(src_hbm.at[wid, g], srcv.at[b], isems[b])
            pltpu.async_copy(dst_hbm.at[wid, g], dstv.at[b], isems[b])

        def idx_wait(g):
            b = g % 2
            pltpu.make_async_copy(src_hbm.at[wid, g], srcv.at[b], isems[b]).wait()
            pltpu.make_async_copy(dst_hbm.at[wid, g], dstv.at[b], isems[b]).wait()

        idx_load(0)
        pltpu.sync_copy(zz_hbm, acc.at[pl.ds(row0, ROWS_PER_TILE)])
        if compute_deg:
            for j in range(CH // 16):
                onesv[pl.ds(j * 16, 16)] = jnp.ones((16,), _f32)

            @pl.when(sid == 0)
            def _():
                pltpu.sync_copy(zzdeg_hbm, accdeg)
        plsc.subcore_barrier()

        for g in range(IG):
            b = g % 2
            idx_wait(g)
            if g + 1 < IG:
                idx_load(g + 1)
            pltpu.async_copy(hn_hbm.at[srcv.at[b, 0]], rows0, gsem0)

            def body(i, carry, b=b):
                c = 2 * i
                pltpu.async_copy(hn_hbm.at[srcv.at[b, c + 1]], rows1, gsem1)
                if compute_deg:
                    pltpu.sync_copy(onesv, accdeg.at[dstv.at[b, c]], add=True)
                pltpu.make_async_copy(hn_hbm.at[srcv.at[b, c]], rows0, gsem0).wait()
                pltpu.sync_copy(rows0, acc.at[dstv.at[b, c]], add=True)
                pltpu.async_copy(hn_hbm.at[srcv.at[b, c + 2]], rows0, gsem0)
                if compute_deg:
                    pltpu.sync_copy(onesv, accdeg.at[dstv.at[b, c + 1]], add=True)
                pltpu.make_async_copy(hn_hbm.at[srcv.at[b, c + 1]], rows1, gsem1).wait()
                pltpu.sync_copy(rows1, acc.at[dstv.at[b, c + 1]], add=True)
                return carry

            lax.fori_loop(0, (GN - 1) // 2, body, 0)
            pltpu.make_async_copy(hn_hbm.at[srcv.at[b, GN - 1]], rows0, gsem0).wait()
            pltpu.sync_copy(rows0, acc.at[dstv.at[b, GN - 1]], add=True)
            if compute_deg:
                pltpu.sync_copy(onesv, accdeg.at[dstv.at[b, GN - 1]], add=True)

        plsc.subcore_barrier()

        @pl.when(cid == 0)
        def _():
            pltpu.sync_copy(acc.at[pl.ds(row0, ROWS_PER_TILE)],
                            out_a.at[pl.ds(row0, ROWS_PER_TILE)])

        @pl.when(cid == 1)
        def _():
            pltpu.sync_copy(acc.at[pl.ds(row0, ROWS_PER_TILE)],
                            out_b.at[pl.ds(row0, ROWS_PER_TILE)])

        if compute_deg:
            @pl.when(sid == 1)
            def _():
                pltpu.sync_copy(accdeg, out_deg.at[pl.ds(cid * NPAD, NPAD)])

    return sagg


_sagg_deg = _make_sagg(True)
_sagg = _make_sagg(False)


def _make_readout():
    mesh = plsc.VectorSubcoreMesh(core_axis_name="c", subcore_axis_name="s",
                                  num_cores=2, num_subcores=16)
    rows = P // NTILES

    @functools.partial(
        pl.kernel,
        out_type=jax.ShapeDtypeStruct((P, D), _f32),
        mesh=mesh,
        scratch_types=[
            pltpu.VMEM((rows,), jnp.int32),
            pltpu.VMEM((rows,), jnp.int32),
            pltpu.VMEM((rows, D), _f32),
            pltpu.VMEM((rows, D), _f32),
            pltpu.SemaphoreType.DMA,
            pltpu.SemaphoreType.DMA,
        ],
    )
    def readout(z_hbm, x1_hbm, x2_hbm, out_hbm, i1, i2, r1, r2, s1, s2):
        cid = lax.axis_index("c")
        sid = lax.axis_index("s")
        base = (cid * 16 + sid) * rows
        pltpu.sync_copy(x1_hbm.at[pl.ds(base, rows)], i1)
        pltpu.sync_copy(x2_hbm.at[pl.ds(base, rows)], i2)
        d1 = pltpu.async_copy(z_hbm.at[i1], r1, s1)
        d2 = pltpu.async_copy(z_hbm.at[i2], r2, s2)
        d1.wait()
        d2.wait()

        def add_row(i, carry):
            for j in range(D // 16):
                sl = pl.ds(j * 16, 16)
                r1[i, sl] = r1[i, sl] + r2[i, sl]
            return carry

        lax.fori_loop(0, rows, add_row, 0)
        pltpu.sync_copy(r1, out_hbm.at[pl.ds(base, rows)])

    return readout


_readout = _make_readout()


def kernel(h, edge_index, x1, x2, W_self0, W_neigh0, b0,
           W_self1, W_neigh1, b1, W2, b2):
    src3d = edge_index[0].reshape(NTILES, IG, GN, CH)
    dst3d = edge_index[1].reshape(NTILES, IG, GN, CH)

    b0r = b0.reshape(1, D)
    b1r = b1.reshape(1, D)
    w2p = jnp.pad(W2, ((0, 0), (0, D - C)))
    b2h = jnp.pad(0.5 * b2, (0, D - C)).reshape(1, D)
    zz = jnp.zeros((ROWS_PER_TILE, D), _f32)
    zzdeg = jnp.zeros((NPAD,), _f32)

    hn0 = _k1(h, W_neigh0)
    agg_a0, agg_b0, deg = _sagg_deg(hn0, src3d, dst3d, zz, zzdeg)
    degp = deg.reshape(2, NPAD // D, D)
    h1, hn1 = _k2(h, agg_a0, agg_b0, degp, W_self0, b0r, W_neigh1)
    agg_a1, agg_b1 = _sagg(hn1, src3d, dst3d, zz)
    z = _k3(h1, agg_a1, agg_b1, degp, W_self1, b1r, w2p, b2h)
    outp = _readout(z, x1, x2)
    return outp[:, :C]


# speedup vs baseline: 2.8406x; 1.0657x over previous
import functools

import jax
import jax.numpy as jnp
from jax import lax
from jax.experimental import pallas as pl
from jax.experimental.pallas import tpu as pltpu
from jax.experimental.pallas import tpu_sc as plsc

N = 10000
E = 320000
D = 128
C = 10
P = 4096

NPAD = 10240
NTILES = 32
EP = E // NTILES
CH = 100
NCHT = EP // CH
IG = 4
GN = NCHT // IG
ROWS_PER_TILE = NPAD // 16
BM = 1024

_f32 = jnp.float32


def _k1_body(h_ref, wn_ref, hn_ref):
    hn_ref[...] = jnp.dot(h_ref[...], wn_ref[...], preferred_element_type=_f32)


def _k1(h, wn):
    grid = (NPAD // BM,)
    return pl.pallas_call(
        _k1_body,
        grid=grid,
        in_specs=[
            pl.BlockSpec((BM, D), lambda i: (i, 0)),
            pl.BlockSpec((D, D), lambda i: (0, 0)),
        ],
        out_specs=pl.BlockSpec((BM, D), lambda i: (i, 0)),
        out_shape=jax.ShapeDtypeStruct((NPAD, D), _f32),
    )(h, wn)


def _dinv_col(degp_blk):
    deg8 = degp_blk[0] + degp_blk[1]
    dinv8 = 1.0 / jnp.maximum(deg8, 1.0)
    lane_of_row = lax.broadcasted_iota(jnp.int32, (BM, D), 0) % D
    lane_ids = lax.broadcasted_iota(jnp.int32, (BM, D), 1)
    onehot = (lane_of_row == lane_ids).astype(_f32)
    r = lax.dot_general(onehot, dinv8, (((1,), (1,)), ((), ())),
                        preferred_element_type=_f32)
    grp = lax.broadcasted_iota(jnp.int32, (BM, 8), 0) // D
    sel = (grp == lax.broadcasted_iota(jnp.int32, (BM, 8), 1)).astype(_f32)
    return jnp.sum(r * sel, axis=1, keepdims=True)


def _k2_body(h_ref, a0_ref, a1_ref, degp_ref, ws_ref, b_ref, wn_ref,
             h1_ref, hn1_ref):
    dinv = _dinv_col(degp_ref[...])
    hs0 = jnp.dot(h_ref[...], ws_ref[...], preferred_element_type=_f32) + b_ref[...]
    h1 = jnp.maximum(hs0 + (a0_ref[...] + a1_ref[...]) * dinv, 0.0)
    h1_ref[...] = h1
    hn1_ref[...] = jnp.dot(h1, wn_ref[...], preferred_element_type=_f32)


def _k2(h, agg_a, agg_b, degp, ws, b, wn):
    grid = (NPAD // BM,)
    return pl.pallas_call(
        _k2_body,
        grid=grid,
        in_specs=[
            pl.BlockSpec((BM, D), lambda i: (i, 0)),
            pl.BlockSpec((BM, D), lambda i: (i, 0)),
            pl.BlockSpec((BM, D), lambda i: (i, 0)),
            pl.BlockSpec((2, 8, D), lambda i: (0, i, 0)),
            pl.BlockSpec((D, D), lambda i: (0, 0)),
            pl.BlockSpec((1, D), lambda i: (0, 0)),
            pl.BlockSpec((D, D), lambda i: (0, 0)),
        ],
        out_specs=[
            pl.BlockSpec((BM, D), lambda i: (i, 0)),
            pl.BlockSpec((BM, D), lambda i: (i, 0)),
        ],
        out_shape=[
            jax.ShapeDtypeStruct((NPAD, D), _f32),
            jax.ShapeDtypeStruct((NPAD, D), _f32),
        ],
    )(h, agg_a, agg_b, degp, ws, b, wn)


def _k3_body(h1_ref, a0_ref, a1_ref, degp_ref, ws_ref, b_ref, w2_ref, b2_ref,
             z_ref):
    dinv = _dinv_col(degp_ref[...])
    hs1 = jnp.dot(h1_ref[...], ws_ref[...], preferred_element_type=_f32) + b_ref[...]
    h2 = jnp.maximum(hs1 + (a0_ref[...] + a1_ref[...]) * dinv, 0.0)
    z_ref[...] = jnp.dot(h2, w2_ref[...], preferred_element_type=_f32) + b2_ref[...]


def _k3(h1, agg_a, agg_b, degp, ws, b, w2p, b2h):
    grid = (NPAD // BM,)
    return pl.pallas_call(
        _k3_body,
        grid=grid,
        in_specs=[
            pl.BlockSpec((BM, D), lambda i: (i, 0)),
            pl.BlockSpec((BM, D), lambda i: (i, 0)),
            pl.BlockSpec((BM, D), lambda i: (i, 0)),
            pl.BlockSpec((2, 8, D), lambda i: (0, i, 0)),
            pl.BlockSpec((D, D), lambda i: (0, 0)),
            pl.BlockSpec((1, D), lambda i: (0, 0)),
            pl.BlockSpec((D, D), lambda i: (0, 0)),
            pl.BlockSpec((1, D), lambda i: (0, 0)),
        ],
        out_specs=pl.BlockSpec((BM, D), lambda i: (i, 0)),
        out_shape=jax.ShapeDtypeStruct((NPAD, D), _f32),
    )(h1, agg_a, agg_b, degp, ws, b, w2p, b2h)


def _make_sagg(compute_deg):
    mesh = plsc.VectorSubcoreMesh(core_axis_name="c", subcore_axis_name="s",
                                  num_cores=2, num_subcores=16)
    out_type = [
        jax.ShapeDtypeStruct((NPAD, D), _f32),
        jax.ShapeDtypeStruct((NPAD, D), _f32),
    ]
    scratch = [
        pltpu.VMEM((2, GN, CH), jnp.int32),
        pltpu.VMEM((2, GN, CH), jnp.int32),
        pltpu.VMEM((CH, D), _f32),
        pltpu.VMEM((CH, D), _f32),
        pltpu.VMEM_SHARED((NPAD, D), _f32),
        pltpu.SemaphoreType.DMA,
        pltpu.SemaphoreType.DMA,
        pltpu.SemaphoreType.DMA,
        pltpu.SemaphoreType.DMA,
    ]
    if compute_deg:
        out_type = out_type + [jax.ShapeDtypeStruct((2 * NPAD,), _f32)]
        scratch = scratch + [
            pltpu.VMEM((CH,), _f32),
            pltpu.VMEM_SHARED((NPAD,), _f32),
        ]

    @functools.partial(pl.kernel, out_type=tuple(out_type), mesh=mesh,
                       scratch_types=scratch)
    def sagg(hn_hbm, src_hbm, dst_hbm, zz_hbm, *rest):
        if compute_deg:
            (zzdeg_hbm, out_a, out_b, out_deg,
             srcv, dstv, rows0, rows1, acc, gsem0, gsem1, isem0, isem1,
             onesv, accdeg) = rest
        else:
            (out_a, out_b,
             srcv, dstv, rows0, rows1, acc, gsem0, gsem1, isem0, isem1) = rest
        cid = lax.axis_index("c")
        sid = lax.axis_index("s")
        wid = cid * 16 + sid
        row0 = sid * ROWS_PER_TILE
        isems = (isem0, isem1)

        def idx_load(g):
            b = g % 2
            pltpu.async_copy(src_hbm.at[wid, g], srcv.at[b], isems[b])
            pltpu.async_copy(dst_hbm.at[wid, g], dstv.at[b], isems[b])

        def idx_wait(g):
            b = g % 2
            pltpu.make_async_copy(src_hbm.at[wid, g], srcv.at[b], isems[b]).wait()
            pltpu.make_async_copy(dst_hbm.at[wid, g], dstv.at[b], isems[b]).wait()

        idx_load(0)
        pltpu.sync_copy(zz_hbm, acc.at[pl.ds(row0, ROWS_PER_TILE)])
        if compute_deg:
            for j in range(CH // 16):
                onesv[pl.ds(j * 16, 16)] = jnp.ones((16,), _f32)

            @pl.when(sid == 0)
            def _():
                pltpu.sync_copy(zzdeg_hbm, accdeg)
        plsc.subcore_barrier()

        for g in range(IG):
            b = g % 2
            idx_wait(g)
            if g + 1 < IG:
                idx_load(g + 1)
            pltpu.async_copy(hn_hbm.at[srcv.at[b, 0]], rows0, gsem0)

            def body(i, carry, b=b):
                c = 2 * i
                pltpu.async_copy(hn_hbm.at[srcv.at[b, c + 1]], rows1, gsem1)
                if compute_deg:
                    pltpu.sync_copy(onesv, accdeg.at[dstv.at[b, c]], add=True)
                pltpu.make_async_copy(hn_hbm.at[srcv.at[b, c]], rows0, gsem0).wait()
                pltpu.sync_copy(rows0, acc.at[dstv.at[b, c]], add=True)
                pltpu.async_copy(hn_hbm.at[srcv.at[b, c + 2]], rows0, gsem0)
                if compute_deg:
                    pltpu.sync_copy(onesv, accdeg.at[dstv.at[b, c + 1]], add=True)
                pltpu.make_async_copy(hn_hbm.at[srcv.at[b, c + 1]], rows1, gsem1).wait()
                pltpu.sync_copy(rows1, acc.at[dstv.at[b, c + 1]], add=True)
                return carry

            lax.fori_loop(0, (GN - 1) // 2, body, 0)
            pltpu.make_async_copy(hn_hbm.at[srcv.at[b, GN - 1]], rows0, gsem0).wait()
            pltpu.sync_copy(rows0, acc.at[dstv.at[b, GN - 1]], add=True)
            if compute_deg:
                pltpu.sync_copy(onesv, accdeg.at[dstv.at[b, GN - 1]], add=True)

        plsc.subcore_barrier()

        @pl.when(cid == 0)
        def _():
            pltpu.sync_copy(acc.at[pl.ds(row0, ROWS_PER_TILE)],
                            out_a.at[pl.ds(row0, ROWS_PER_TILE)])

        @pl.when(cid == 1)
        def _():
            pltpu.sync_copy(acc.at[pl.ds(row0, ROWS_PER_TILE)],
                            out_b.at[pl.ds(row0, ROWS_PER_TILE)])

        if compute_deg:
            @pl.when(sid == 1)
            def _():
                pltpu.sync_copy(accdeg, out_deg.at[pl.ds(cid * NPAD, NPAD)])

    return sagg


_sagg_deg = _make_sagg(True)
_sagg = _make_sagg(False)


def _make_readout():
    mesh = plsc.VectorSubcoreMesh(core_axis_name="c", subcore_axis_name="s",
                                  num_cores=2, num_subcores=16)
    rows = P // NTILES

    @functools.partial(
        pl.kernel,
        out_type=jax.ShapeDtypeStruct((P, D), _f32),
        mesh=mesh,
        scratch_types=[
            pltpu.VMEM((rows,), jnp.int32),
            pltpu.VMEM((rows,), jnp.int32),
            pltpu.VMEM((rows, D), _f32),
            pltpu.VMEM((rows, D), _f32),
            pltpu.SemaphoreType.DMA,
            pltpu.SemaphoreType.DMA,
        ],
    )
    def readout(z_hbm, x1_hbm, x2_hbm, out_hbm, i1, i2, r1, r2, s1, s2):
        cid = lax.axis_index("c")
        sid = lax.axis_index("s")
        base = (cid * 16 + sid) * rows
        pltpu.sync_copy(x1_hbm.at[pl.ds(base, rows)], i1)
        pltpu.sync_copy(x2_hbm.at[pl.ds(base, rows)], i2)
        d1 = pltpu.async_copy(z_hbm.at[i1], r1, s1)
        d2 = pltpu.async_copy(z_hbm.at[i2], r2, s2)
        d1.wait()
        d2.wait()

        def add_row(i, carry):
            for j in range(D // 16):
                sl = pl.ds(j * 16, 16)
                r1[i, sl] = r1[i, sl] + r2[i, sl]
            return carry

        lax.fori_loop(0, rows, add_row, 0)
        pltpu.sync_copy(r1, out_hbm.at[pl.ds(base, rows)])

    return readout


_readout = _make_readout()


def kernel(h, edge_index, x1, x2, W_self0, W_neigh0, b0,
           W_self1, W_neigh1, b1, W2, b2):
    src3d = edge_index[0].reshape(NTILES, IG, GN, CH)
    dst3d = edge_index[1].reshape(NTILES, IG, GN, CH)

    b0r = b0.reshape(1, D)
    b1r = b1.reshape(1, D)
    w2p = jnp.pad(W2, ((0, 0), (0, D - C)))
    b2h = jnp.pad(0.5 * b2, (0, D - C)).reshape(1, D)
    zz = jnp.zeros((ROWS_PER_TILE, D), _f32)
    zzdeg = jnp.zeros((NPAD,), _f32)

    hn0 = _k1(h, W_neigh0)
    agg_a0, agg_b0, deg = _sagg_deg(hn0, src3d, dst3d, zz, zzdeg)
    degp = deg.reshape(2, NPAD // D, D)
    h1, hn1 = _k2(h, agg_a0, agg_b0, degp, W_self0, b0r, W_neigh1)
    agg_a1, agg_b1 = _sagg(hn1, src3d, dst3d, zz)
    z = _k3(h1, agg_a1, agg_b1, degp, W_self1, b1r, w2p, b2h)
    outp = _readout(z, x1, x2)
    return outp[:, :C]
